# baseline (speedup 1.0000x reference)
import functools

import jax
import jax.numpy as jnp
from jax import lax
from jax.experimental import pallas as pl
from jax.experimental.pallas import tpu as pltpu
from jax.experimental.pallas import tpu_sc as plsc

NUM_ROWS = 1000000
DIM = 64
NMAJ = 16384
NIDX = 50

_info = plsc.get_sparse_core_info()
NC, NS = _info.num_cores, _info.num_subcores
NW = NC * NS
MAJ_PER_W = NMAJ // NW
NBUF = 4
NGRP = MAJ_PER_W // NBUF


def _sc_gather(table, idx):
    mesh = plsc.VectorSubcoreMesh(core_axis_name="c", subcore_axis_name="s")

    @functools.partial(
        pl.kernel,
        out_type=jax.ShapeDtypeStruct((NMAJ, NIDX, DIM), jnp.float32),
        mesh=mesh,
        scratch_types=[
            pltpu.VMEM((MAJ_PER_W, NIDX), jnp.int32),
            pltpu.VMEM((NIDX, DIM), jnp.float32),
            pltpu.VMEM((NIDX, DIM), jnp.float32),
            pltpu.VMEM((NIDX, DIM), jnp.float32),
            pltpu.VMEM((NIDX, DIM), jnp.float32),
            pltpu.SemaphoreType.DMA,
            pltpu.SemaphoreType.DMA,
            pltpu.SemaphoreType.DMA,
            pltpu.SemaphoreType.DMA,
            pltpu.SemaphoreType.DMA,
            pltpu.SemaphoreType.DMA,
            pltpu.SemaphoreType.DMA,
            pltpu.SemaphoreType.DMA,
        ],
        compiler_params=pltpu.CompilerParams(use_tc_tiling_on_sc=False),
    )
    def k(table_hbm, idx_hbm, out_hbm, idx_v,
          r0, r1, r2, r3, g0, g1, g2, g3, s0, s1, s2, s3):
        wid = lax.axis_index("s") * NC + lax.axis_index("c")
        base = wid * MAJ_PER_W
        pltpu.sync_copy(idx_hbm.at[pl.ds(base, MAJ_PER_W)], idx_v)

        rows = (r0, r1, r2, r3)
        gsem = (g0, g1, g2, g3)
        ssem = (s0, s1, s2, s3)

        def g_start(c, b):
            pltpu.async_copy(
                table_hbm.at[idx_v.at[c]], rows[b], gsem[b]
            )

        def g_wait(b):
            pltpu.make_async_copy(
                table_hbm.at[idx_v.at[0]], rows[b], gsem[b]
            ).wait()

        def s_start(c, b):
            pltpu.async_copy(rows[b], out_hbm.at[base + c], ssem[b])

        def s_wait(b):
            pltpu.make_async_copy(
                rows[b], out_hbm.at[base], ssem[b]
            ).wait()

        g_start(0, 0)
        g_start(1, 1)

        g_start(2, 2)
        g_wait(0)
        s_start(0, 0)
        g_start(3, 3)
        g_wait(1)
        s_start(1, 1)
        s_wait(0)
        g_start(4, 0)
        g_wait(2)
        s_start(2, 2)
        s_wait(1)
        g_start(5, 1)
        g_wait(3)
        s_start(3, 3)

        def body(g, carry):
            c0 = g * NBUF
            for b in range(NBUF):
                c = c0 + b
                bf = (b + 2) % NBUF
                s_wait(bf)
                g_start(c + 2, bf)
                g_wait(b)
                s_start(c, b)
            return carry

        lax.fori_loop(1, NGRP - 1, body, 0)

        cl = MAJ_PER_W - NBUF
        s_wait(2)
        g_start(MAJ_PER_W - 2, 2)
        g_wait(0)
        s_start(cl, 0)
        s_wait(3)
        g_start(MAJ_PER_W - 1, 3)
        g_wait(1)
        s_start(cl + 1, 1)
        s_wait(0)
        g_wait(2)
        s_start(MAJ_PER_W - 2, 2)
        s_wait(1)
        g_wait(3)
        s_start(MAJ_PER_W - 1, 3)
        s_wait(2)
        s_wait(3)

    return k(table, idx)


def kernel(token_ids, weights):
    return _sc_gather(weights, token_ids.astype(jnp.int32))


# speedup vs baseline: 1.2687x; 1.2687x over previous
import functools

import jax
import jax.numpy as jnp
from jax import lax
from jax.experimental import pallas as pl
from jax.experimental.pallas import tpu as pltpu
from jax.experimental.pallas import tpu_sc as plsc

NUM_ROWS = 1000000
DIM = 64
NMAJ = 16384
NIDX = 50

_info = plsc.get_sparse_core_info()
NC, NS = _info.num_cores, _info.num_subcores
NW = NC * NS
MAJ_PER_W = NMAJ // NW
NBUF = 4
NGRP = MAJ_PER_W // NBUF


def _sc_gather(table, idx):
    mesh = plsc.VectorSubcoreMesh(core_axis_name="c", subcore_axis_name="s")

    @functools.partial(
        pl.kernel,
        out_type=jax.ShapeDtypeStruct((NMAJ, NIDX, DIM), jnp.float32),
        mesh=mesh,
        scratch_types=[
            pltpu.VMEM((MAJ_PER_W, NIDX), jnp.int32),
            pltpu.VMEM((NIDX, DIM), jnp.float32),
            pltpu.VMEM((NIDX, DIM), jnp.float32),
            pltpu.VMEM((NIDX, DIM), jnp.float32),
            pltpu.VMEM((NIDX, DIM), jnp.float32),
            pltpu.SemaphoreType.DMA,
            pltpu.SemaphoreType.DMA,
            pltpu.SemaphoreType.DMA,
            pltpu.SemaphoreType.DMA,
            pltpu.SemaphoreType.DMA,
            pltpu.SemaphoreType.DMA,
            pltpu.SemaphoreType.DMA,
            pltpu.SemaphoreType.DMA,
        ],
        compiler_params=pltpu.CompilerParams(use_tc_tiling_on_sc=False),
    )
    def k(table_hbm, idx_hbm, out_hbm, idx_v,
          r0, r1, r2, r3, g0, g1, g2, g3, s0, s1, s2, s3):
        wid = lax.axis_index("s") * NC + lax.axis_index("c")
        base = wid * MAJ_PER_W
        pltpu.sync_copy(idx_hbm.at[pl.ds(base, MAJ_PER_W)], idx_v)

        rows = (r0, r1, r2, r3)
        gsem = (g0, g1, g2, g3)
        ssem = (s0, s1, s2, s3)

        def g_start(c, b):
            pltpu.async_copy(
                table_hbm.at[idx_v.at[c]], rows[b], gsem[b]
            )

        def g_wait(b):
            pltpu.make_async_copy(
                table_hbm.at[idx_v.at[0]], rows[b], gsem[b]
            ).wait()

        def s_start(c, b):
            pltpu.async_copy(rows[b], out_hbm.at[base + c], ssem[b])

        def s_wait(b):
            pltpu.make_async_copy(
                rows[b], out_hbm.at[base], ssem[b]
            ).wait()

        g_start(0, 0)
        g_start(1, 1)

        g_start(2, 2)
        g_wait(0)
        s_start(0, 0)
        g_start(3, 3)
        g_wait(1)
        s_start(1, 1)
        s_wait(0)
        g_start(4, 0)
        g_wait(2)
        s_start(2, 2)
        s_wait(1)
        g_start(5, 1)
        g_wait(3)
        s_start(3, 3)

        def body(g, carry):
            c0 = g * NBUF
            for b in range(NBUF):
                c = c0 + b
                bf = (b + 2) % NBUF
                s_wait(bf)
                g_start(c + 2, bf)
                g_wait(b)
                s_start(c, b)
            return carry

        lax.fori_loop(1, NGRP - 1, body, 0)

        cl = MAJ_PER_W - NBUF
        s_wait(2)
        g_start(MAJ_PER_W - 2, 2)
        g_wait(0)
        s_start(cl, 0)
        s_wait(3)
        g_start(MAJ_PER_W - 1, 3)
        g_wait(1)
        s_start(cl + 1, 1)
        s_wait(0)
        g_wait(2)
        s_start(MAJ_PER_W - 2, 2)
        s_wait(1)
        g_wait(3)
        s_start(MAJ_PER_W - 1, 3)
        s_wait(2)
        s_wait(3)

    return k(table, idx)


TC_BM = 128
ROW128 = NIDX * DIM // 128


def _tc_transpose_kernel(x_ref, o_ref):
    x = x_ref[...].reshape(TC_BM, ROW128, 128)
    for qr in range(ROW128):
        o_ref[2 * qr:2 * qr + 2] = (
            x[:, qr, :].T.reshape(2, DIM, TC_BM)
        )


def _tc_transpose(x128):
    grid = NMAJ // TC_BM
    return pl.pallas_call(
        _tc_transpose_kernel,
        grid=(grid,),
        in_specs=[pl.BlockSpec((TC_BM * ROW128, 128), lambda i: (i, 0))],
        out_specs=pl.BlockSpec((NIDX, DIM, TC_BM), lambda i: (0, 0, i)),
        out_shape=jax.ShapeDtypeStruct((NIDX, DIM, NMAJ), jnp.float32),
    )(x128)


def kernel(token_ids, weights):
    out = _sc_gather(weights, token_ids.astype(jnp.int32))
    out_t = _tc_transpose(out.reshape(NMAJ * NIDX * DIM // 128, 128))
    return jnp.transpose(out_t, (2, 0, 1))


# speedup vs baseline: 1.3234x; 1.0431x over previous
import functools

import jax
import jax.numpy as jnp
from jax import lax
from jax.experimental import pallas as pl
from jax.experimental.pallas import tpu as pltpu
from jax.experimental.pallas import tpu_sc as plsc

NUM_ROWS = 1000000
DIM = 64
NMAJ = 16384
NIDX = 50

_info = plsc.get_sparse_core_info()
NC, NS = _info.num_cores, _info.num_subcores
NW = NC * NS
MAJ_PER_W = NMAJ // NW
NBUF = 4
NGRP = MAJ_PER_W // NBUF


def _sc_gather(table, idx):
    mesh = plsc.VectorSubcoreMesh(core_axis_name="c", subcore_axis_name="s")

    @functools.partial(
        pl.kernel,
        out_type=jax.ShapeDtypeStruct((NMAJ, NIDX, DIM), jnp.float32),
        mesh=mesh,
        scratch_types=[
            pltpu.VMEM((MAJ_PER_W, NIDX), jnp.int32),
            pltpu.VMEM((NIDX, DIM), jnp.float32),
            pltpu.VMEM((NIDX, DIM), jnp.float32),
            pltpu.VMEM((NIDX, DIM), jnp.float32),
            pltpu.VMEM((NIDX, DIM), jnp.float32),
            pltpu.SemaphoreType.DMA,
            pltpu.SemaphoreType.DMA,
            pltpu.SemaphoreType.DMA,
            pltpu.SemaphoreType.DMA,
            pltpu.SemaphoreType.DMA,
            pltpu.SemaphoreType.DMA,
            pltpu.SemaphoreType.DMA,
            pltpu.SemaphoreType.DMA,
        ],
        compiler_params=pltpu.CompilerParams(use_tc_tiling_on_sc=False),
    )
    def k(table_hbm, idx_hbm, out_hbm, idx_v,
          r0, r1, r2, r3, g0, g1, g2, g3, s0, s1, s2, s3):
        wid = lax.axis_index("s") * NC + lax.axis_index("c")
        base = wid * MAJ_PER_W
        pltpu.sync_copy(idx_hbm.at[pl.ds(base, MAJ_PER_W)], idx_v)

        rows = (r0, r1, r2, r3)
        gsem = (g0, g1, g2, g3)
        ssem = (s0, s1, s2, s3)

        def g_start(c, b):
            pltpu.async_copy(
                table_hbm.at[idx_v.at[c]], rows[b], gsem[b]
            )

        def g_wait(b):
            pltpu.make_async_copy(
                table_hbm.at[idx_v.at[0]], rows[b], gsem[b]
            ).wait()

        def s_start(c, b):
            pltpu.async_copy(rows[b], out_hbm.at[base + c], ssem[b])

        def s_wait(b):
            pltpu.make_async_copy(
                rows[b], out_hbm.at[base], ssem[b]
            ).wait()

        g_start(0, 0)
        g_start(1, 1)

        g_start(2, 2)
        g_wait(0)
        s_start(0, 0)
        g_start(3, 3)
        g_wait(1)
        s_start(1, 1)
        s_wait(0)
        g_start(4, 0)
        g_wait(2)
        s_start(2, 2)
        s_wait(1)
        g_start(5, 1)
        g_wait(3)
        s_start(3, 3)

        def body(g, carry):
            c0 = g * NBUF
            for b in range(NBUF):
                c = c0 + b
                bf = (b + 2) % NBUF
                s_wait(bf)
                g_start(c + 2, bf)
                g_wait(b)
                s_start(c, b)
            return carry

        lax.fori_loop(1, NGRP - 1, body, 0)

        cl = MAJ_PER_W - NBUF
        s_wait(2)
        g_start(MAJ_PER_W - 2, 2)
        g_wait(0)
        s_start(cl, 0)
        s_wait(3)
        g_start(MAJ_PER_W - 1, 3)
        g_wait(1)
        s_start(cl + 1, 1)
        s_wait(0)
        g_wait(2)
        s_start(MAJ_PER_W - 2, 2)
        s_wait(1)
        g_wait(3)
        s_start(MAJ_PER_W - 1, 3)
        s_wait(2)
        s_wait(3)

    return k(table, idx)


TC_BM = 128
ROW128 = NIDX * DIM // 128


def _tc_transpose_kernel(x_ref, o_ref):
    x = x_ref[...].reshape(TC_BM, ROW128, 128)
    for qr in range(ROW128):
        o_ref[2 * qr:2 * qr + 2] = (
            x[:, qr, :].T.reshape(2, DIM, TC_BM)
        )


def _tc_transpose(x128):
    grid = NMAJ // TC_BM
    return pl.pallas_call(
        _tc_transpose_kernel,
        grid=(grid,),
        in_specs=[pl.BlockSpec((TC_BM * ROW128, 128), lambda i: (i, 0))],
        out_specs=pl.BlockSpec((NIDX, DIM, TC_BM), lambda i: (0, 0, i)),
        out_shape=jax.ShapeDtypeStruct((NIDX, DIM, NMAJ), jnp.float32),
    )(x128)


TC_BN = 2048


def _tc_prep_kernel(x_ref, o_ref):
    y = jnp.transpose(x_ref[...], (1, 0)).reshape(TC_BN // 2, 2, DIM)
    o_ref[:, 0:DIM] = y[:, 0, :]
    o_ref[:, DIM:2 * DIM] = y[:, 1, :]


def _tc_prep(wt):
    grid = (NUM_ROWS + TC_BN - 1) // TC_BN
    return pl.pallas_call(
        _tc_prep_kernel,
        grid=(grid,),
        in_specs=[pl.BlockSpec((DIM, TC_BN), lambda j: (0, j))],
        out_specs=pl.BlockSpec((TC_BN // 2, 128), lambda j: (j, 0)),
        out_shape=jax.ShapeDtypeStruct((NUM_ROWS // 2, 128), jnp.float32),
    )(wt)


def kernel(token_ids, weights):
    table = _tc_prep(weights.T).reshape(NUM_ROWS, DIM)
    out = _sc_gather(table, token_ids.astype(jnp.int32))
    out_t = _tc_transpose(out.reshape(NMAJ * NIDX * DIM // 128, 128))
    return jnp.transpose(out_t, (2, 0, 1))


# speedup vs baseline: 1.6355x; 1.2359x over previous
import functools

import jax
import jax.numpy as jnp
from jax import lax
from jax.experimental import pallas as pl
from jax.experimental.pallas import tpu as pltpu
from jax.experimental.pallas import tpu_sc as plsc

NUM_ROWS = 1000000
DIM = 64
NMAJ = 16384
NIDX = 50

_info = plsc.get_sparse_core_info()
NC, NS = _info.num_cores, _info.num_subcores
NW = NC * NS
MAJ_PER_W = NMAJ // NW
NBUF = 4
NGRP = MAJ_PER_W // NBUF


def _sc_gather(table, idx):
    mesh = plsc.VectorSubcoreMesh(core_axis_name="c", subcore_axis_name="s")

    @functools.partial(
        pl.kernel,
        out_type=jax.ShapeDtypeStruct((NMAJ, NIDX, DIM), jnp.float32),
        mesh=mesh,
        scratch_types=[
            pltpu.VMEM((MAJ_PER_W, NIDX), jnp.int32),
            pltpu.VMEM((NIDX, DIM), jnp.float32),
            pltpu.VMEM((NIDX, DIM), jnp.float32),
            pltpu.VMEM((NIDX, DIM), jnp.float32),
            pltpu.VMEM((NIDX, DIM), jnp.float32),
            pltpu.SemaphoreType.DMA,
            pltpu.SemaphoreType.DMA,
            pltpu.SemaphoreType.DMA,
            pltpu.SemaphoreType.DMA,
            pltpu.SemaphoreType.DMA,
            pltpu.SemaphoreType.DMA,
            pltpu.SemaphoreType.DMA,
            pltpu.SemaphoreType.DMA,
        ],
        compiler_params=pltpu.CompilerParams(use_tc_tiling_on_sc=False),
    )
    def k(table_hbm, idx_hbm, out_hbm, idx_v,
          r0, r1, r2, r3, g0, g1, g2, g3, s0, s1, s2, s3):
        wid = lax.axis_index("s") * NC + lax.axis_index("c")
        base = wid * MAJ_PER_W
        pltpu.sync_copy(idx_hbm.at[pl.ds(base, MAJ_PER_W)], idx_v)

        rows = (r0, r1, r2, r3)
        gsem = (g0, g1, g2, g3)
        ssem = (s0, s1, s2, s3)

        def g_start(c, b):
            pltpu.async_copy(
                table_hbm.at[idx_v.at[c]], rows[b], gsem[b]
            )

        def g_wait(b):
            pltpu.make_async_copy(
                table_hbm.at[idx_v.at[0]], rows[b], gsem[b]
            ).wait()

        def s_start(c, b):
            pltpu.async_copy(rows[b], out_hbm.at[base + c], ssem[b])

        def s_wait(b):
            pltpu.make_async_copy(
                rows[b], out_hbm.at[base], ssem[b]
            ).wait()

        g_start(0, 0)
        g_start(1, 1)

        g_start(2, 2)
        g_wait(0)
        s_start(0, 0)
        g_start(3, 3)
        g_wait(1)
        s_start(1, 1)
        s_wait(0)
        g_start(4, 0)
        g_wait(2)
        s_start(2, 2)
        s_wait(1)
        g_start(5, 1)
        g_wait(3)
        s_start(3, 3)

        def body(g, carry):
            c0 = g * NBUF
            for b in range(NBUF):
                c = c0 + b
                bf = (b + 2) % NBUF
                s_wait(bf)
                g_start(c + 2, bf)
                g_wait(b)
                s_start(c, b)
            return carry

        lax.fori_loop(1, NGRP - 1, body, 0)

        cl = MAJ_PER_W - NBUF
        s_wait(2)
        g_start(MAJ_PER_W - 2, 2)
        g_wait(0)
        s_start(cl, 0)
        s_wait(3)
        g_start(MAJ_PER_W - 1, 3)
        g_wait(1)
        s_start(cl + 1, 1)
        s_wait(0)
        g_wait(2)
        s_start(MAJ_PER_W - 2, 2)
        s_wait(1)
        g_wait(3)
        s_start(MAJ_PER_W - 1, 3)
        s_wait(2)
        s_wait(3)

    return k(table, idx)


TC_BM = 256
ROW128 = NIDX * DIM // 128


def _tc_transpose_kernel(x_ref, o_ref):
    x = x_ref[...].reshape(TC_BM, ROW128, 128)
    for qr in range(ROW128):
        o_ref[2 * qr:2 * qr + 2] = (
            x[:, qr, :].T.reshape(2, DIM, TC_BM)
        )


def _tc_transpose(x128):
    grid = NMAJ // TC_BM
    return pl.pallas_call(
        _tc_transpose_kernel,
        grid=(grid,),
        in_specs=[pl.BlockSpec((TC_BM * ROW128, 128), lambda i: (i, 0))],
        out_specs=pl.BlockSpec((NIDX, DIM, TC_BM), lambda i: (0, 0, i)),
        out_shape=jax.ShapeDtypeStruct((NIDX, DIM, NMAJ), jnp.float32),
    )(x128)


TC_BN = 8192


def _tc_prep_kernel(x_ref, o_ref):
    y = jnp.transpose(x_ref[...], (1, 0)).reshape(TC_BN // 2, 2, DIM)
    o_ref[:, 0:DIM] = y[:, 0, :]
    o_ref[:, DIM:2 * DIM] = y[:, 1, :]


def _tc_prep(wt):
    grid = (NUM_ROWS + TC_BN - 1) // TC_BN
    return pl.pallas_call(
        _tc_prep_kernel,
        grid=(grid,),
        in_specs=[pl.BlockSpec((DIM, TC_BN), lambda j: (0, j))],
        out_specs=pl.BlockSpec((TC_BN // 2, 128), lambda j: (j, 0)),
        out_shape=jax.ShapeDtypeStruct((NUM_ROWS // 2, 128), jnp.float32),
    )(wt)


def kernel(token_ids, weights):
    table = _tc_prep(weights.T).reshape(NUM_ROWS, DIM)
    out = _sc_gather(table, token_ids.astype(jnp.int32))
    out_t = _tc_transpose(out.reshape(NMAJ * NIDX * DIM // 128, 128))
    return jnp.transpose(out_t, (2, 0, 1))


# speedup vs baseline: 1.6868x; 1.0314x over previous
import functools

import jax
import jax.numpy as jnp
from jax import lax
from jax.experimental import pallas as pl
from jax.experimental.pallas import tpu as pltpu
from jax.experimental.pallas import tpu_sc as plsc

NUM_ROWS = 1000000
DIM = 64
NMAJ = 16384
NIDX = 50

_info = plsc.get_sparse_core_info()
NC, NS = _info.num_cores, _info.num_subcores
NW = NC * NS
MAJ_PER_W = NMAJ // NW
NBUF = 4
NGRP = MAJ_PER_W // NBUF


def _sc_gather(table, idx):
    mesh = plsc.VectorSubcoreMesh(core_axis_name="c", subcore_axis_name="s")

    @functools.partial(
        pl.kernel,
        out_type=jax.ShapeDtypeStruct((NMAJ, NIDX, DIM), jnp.float32),
        mesh=mesh,
        scratch_types=[
            pltpu.VMEM((MAJ_PER_W, NIDX), jnp.int32),
            pltpu.VMEM((NIDX, DIM), jnp.float32),
            pltpu.VMEM((NIDX, DIM), jnp.float32),
            pltpu.VMEM((NIDX, DIM), jnp.float32),
            pltpu.VMEM((NIDX, DIM), jnp.float32),
            pltpu.SemaphoreType.DMA,
            pltpu.SemaphoreType.DMA,
            pltpu.SemaphoreType.DMA,
            pltpu.SemaphoreType.DMA,
            pltpu.SemaphoreType.DMA,
            pltpu.SemaphoreType.DMA,
            pltpu.SemaphoreType.DMA,
            pltpu.SemaphoreType.DMA,
        ],
        compiler_params=pltpu.CompilerParams(use_tc_tiling_on_sc=False),
    )
    def k(table_hbm, idx_hbm, out_hbm, idx_v,
          r0, r1, r2, r3, g0, g1, g2, g3, s0, s1, s2, s3):
        wid = lax.axis_index("s") * NC + lax.axis_index("c")
        base = wid * MAJ_PER_W
        pltpu.sync_copy(idx_hbm.at[pl.ds(base, MAJ_PER_W)], idx_v)

        rows = (r0, r1, r2, r3)
        gsem = (g0, g1, g2, g3)
        ssem = (s0, s1, s2, s3)

        def g_start(c, b):
            pltpu.async_copy(
                table_hbm.at[idx_v.at[c]], rows[b], gsem[b]
            )

        def g_wait(b):
            pltpu.make_async_copy(
                table_hbm.at[idx_v.at[0]], rows[b], gsem[b]
            ).wait()

        def s_start(c, b):
            pltpu.async_copy(rows[b], out_hbm.at[base + c], ssem[b])

        def s_wait(b):
            pltpu.make_async_copy(
                rows[b], out_hbm.at[base], ssem[b]
            ).wait()

        g_start(0, 0)
        g_start(1, 1)

        g_start(2, 2)
        g_wait(0)
        s_start(0, 0)
        g_start(3, 3)
        g_wait(1)
        s_start(1, 1)
        s_wait(0)
        g_start(4, 0)
        g_wait(2)
        s_start(2, 2)
        s_wait(1)
        g_start(5, 1)
        g_wait(3)
        s_start(3, 3)

        def body(g, carry):
            c0 = g * NBUF
            for b in range(NBUF):
                c = c0 + b
                bf = (b + 2) % NBUF
                s_wait(bf)
                g_start(c + 2, bf)
                g_wait(b)
                s_start(c, b)
            return carry

        lax.fori_loop(1, NGRP - 1, body, 0)

        cl = MAJ_PER_W - NBUF
        s_wait(2)
        g_start(MAJ_PER_W - 2, 2)
        g_wait(0)
        s_start(cl, 0)
        s_wait(3)
        g_start(MAJ_PER_W - 1, 3)
        g_wait(1)
        s_start(cl + 1, 1)
        s_wait(0)
        g_wait(2)
        s_start(MAJ_PER_W - 2, 2)
        s_wait(1)
        g_wait(3)
        s_start(MAJ_PER_W - 1, 3)
        s_wait(2)
        s_wait(3)

    return k(table, idx)


TC_BM = 512
ROW128 = NIDX * DIM // 128


def _tc_transpose_kernel(x_ref, o_ref):
    x = x_ref[...].reshape(TC_BM, ROW128, 128)
    for qr in range(ROW128):
        o_ref[2 * qr:2 * qr + 2] = (
            x[:, qr, :].T.reshape(2, DIM, TC_BM)
        )


def _tc_transpose(x128):
    grid = NMAJ // TC_BM
    return pl.pallas_call(
        _tc_transpose_kernel,
        grid=(grid,),
        in_specs=[pl.BlockSpec((TC_BM * ROW128, 128), lambda i: (i, 0))],
        out_specs=pl.BlockSpec((NIDX, DIM, TC_BM), lambda i: (0, 0, i)),
        out_shape=jax.ShapeDtypeStruct((NIDX, DIM, NMAJ), jnp.float32),
    )(x128)


TC_BN = 16384


def _tc_prep_kernel(x_ref, o_ref):
    y = jnp.transpose(x_ref[...], (1, 0)).reshape(TC_BN // 2, 2, DIM)
    o_ref[:, 0:DIM] = y[:, 0, :]
    o_ref[:, DIM:2 * DIM] = y[:, 1, :]


def _tc_prep(wt):
    grid = (NUM_ROWS + TC_BN - 1) // TC_BN
    return pl.pallas_call(
        _tc_prep_kernel,
        grid=(grid,),
        in_specs=[pl.BlockSpec((DIM, TC_BN), lambda j: (0, j))],
        out_specs=pl.BlockSpec((TC_BN // 2, 128), lambda j: (j, 0)),
        out_shape=jax.ShapeDtypeStruct((NUM_ROWS // 2, 128), jnp.float32),
    )(wt)


def kernel(token_ids, weights):
    table = _tc_prep(weights.T).reshape(NUM_ROWS, DIM)
    out = _sc_gather(table, token_ids.astype(jnp.int32))
    out_t = _tc_transpose(out.reshape(NMAJ * NIDX * DIM // 128, 128))
    return jnp.transpose(out_t, (2, 0, 1))
